# baseline (speedup 1.0000x reference)
import dataclasses
import functools

import jax
import jax.numpy as jnp
from jax import lax
from jax.experimental import pallas as pl
from jax.experimental.pallas import tpu as pltpu
from jax.experimental.pallas import tpu_sc as plsc

N = 10000
E = 320000
D = 128
L = 16

NC = 2
NS = 16
NW = NC * NS
CHUNK = 128
NCHUNK = 80
EW = CHUNK * NCHUNK
EPAD = NW * EW + 2 * CHUNK
NP = 10240
RT = NP // NS

_mesh = plsc.VectorSubcoreMesh(core_axis_name="c", subcore_axis_name="s")


def _sc_body(x_hbm, src_hbm, dst_hbm, z_hbm, out_hbm, cnt_hbm, hist_hbm,
             sb0, db0, sb1, db1, rows0, rows1, hist_v, tmp_v,
             si0, si1, sg0, sg1, acc_sh):
    cid = lax.axis_index("c")
    sid = lax.axis_index("s")
    wid = cid * NS + sid
    row0 = sid * RT

    sb = (sb0, sb1)
    db = (db0, db1)
    rows = (rows0, rows1)
    si = (si0, si1)
    sg = (sg0, sg1)

    pltpu.sync_copy(z_hbm, rows0)
    for k in range(RT // CHUNK):
        pltpu.sync_copy(rows0, acc_sh.at[pl.ds(row0 + k * CHUNK, CHUNK)])

    @pl.loop(0, NP // L)
    def _(i):
        hist_v[pl.ds(i * L, L)] = jnp.zeros((L,), jnp.float32)

    plsc.subcore_barrier()

    base = wid * EW

    @pl.loop(0, NCHUNK)
    def _(j):
        off = base + j * CHUNK
        pltpu.sync_copy(src_hbm.at[pl.ds(off, CHUNK)], sb0)
        pltpu.sync_copy(dst_hbm.at[pl.ds(off, CHUNK)], db0)
        pltpu.sync_copy(x_hbm.at[sb0], rows0)
        pltpu.sync_copy(rows0, acc_sh.at[db0], add=True)
        for k in range(CHUNK // L):
            idx = db0[pl.ds(k * L, L)]
            cnts, last = plsc.scan_count(idx)
            plsc.addupdate_scatter(hist_v, [idx],
                                   cnts.astype(jnp.float32), mask=last)

    pltpu.sync_copy(hist_v, hist_hbm.at[pl.ds(wid * NP, NP)])
    plsc.subcore_barrier()

    nko = RT // CHUNK

    def _osl(k):
        return pl.ds(row0 + k * CHUNK, CHUNK)

    pltpu.async_copy(acc_sh.at[_osl(0)], rows0, sg0)
    for k in range(nko):
        rp = rows[k % 2]
        rq = rows[1 - k % 2]
        sp = sg[k % 2]
        sq = sg[1 - k % 2]
        wp = si[k % 2]
        wq = si[1 - k % 2]
        pltpu.make_async_copy(acc_sh.at[_osl(k)], rp, sp).wait()
        if k + 1 < nko:
            if k >= 1:
                pltpu.make_async_copy(rq, out_hbm.at[cid, _osl(k - 1)],
                                      wq).wait()
            pltpu.async_copy(acc_sh.at[_osl(k + 1)], rq, sq)
        pltpu.async_copy(rp, out_hbm.at[cid, _osl(k)], wp)
    pltpu.make_async_copy(rows[(nko - 2) % 2],
                          out_hbm.at[cid, _osl(nko - 2)],
                          si[(nko - 2) % 2]).wait()
    pltpu.make_async_copy(rows[(nko - 1) % 2],
                          out_hbm.at[cid, _osl(nko - 1)],
                          si[(nko - 1) % 2]).wait()

    lane_iota = jax.lax.iota(jnp.int32, L)
    zeros_i = jnp.zeros((L,), jnp.int32)

    def _hsl(j, k):
        return pl.ds((cid * NS + j) * NP + row0 + k * CHUNK, CHUNK)

    def _fire(k):
        for j in range(NS):
            pltpu.async_copy(hist_hbm.at[_hsl(j, k)],
                             tmp_v.at[pl.ds(j * CHUNK, CHUNK)], sg0)

    def _drain(k):
        for j in range(NS):
            pltpu.make_async_copy(hist_hbm.at[_hsl(j, k)],
                                  tmp_v.at[pl.ds(j * CHUNK, CHUNK)],
                                  sg0).wait()

    _fire(0)
    for k in range(RT // CHUNK):
        _drain(k)

        @pl.loop(0, CHUNK // L)
        def _(b):
            tot = jnp.zeros((L,), jnp.float32)
            for j in range(NS):
                tot += tmp_v[pl.ds(j * CHUNK + b * L, L)]
            plsc.store_scatter(rows0, [b * L + lane_iota, zeros_i], tot)

        if k + 1 < RT // CHUNK:
            _fire(k + 1)
        pltpu.sync_copy(rows0, cnt_hbm.at[cid, _osl(k)])


def _make_sc_pass():
    out_type = (jax.ShapeDtypeStruct((NC, NP, D), jnp.float32),
                jax.ShapeDtypeStruct((NC, NP, D), jnp.float32),
                jax.ShapeDtypeStruct((NW * NP,), jnp.float32))
    scratch = [
        pltpu.VMEM((CHUNK,), jnp.int32),
        pltpu.VMEM((CHUNK,), jnp.int32),
        pltpu.VMEM((CHUNK,), jnp.int32),
        pltpu.VMEM((CHUNK,), jnp.int32),
        pltpu.VMEM((CHUNK, D), jnp.float32),
        pltpu.VMEM((CHUNK, D), jnp.float32),
        pltpu.VMEM((NP,), jnp.float32),
        pltpu.VMEM((NS * CHUNK,), jnp.float32),
        pltpu.SemaphoreType.DMA,
        pltpu.SemaphoreType.DMA,
        pltpu.SemaphoreType.DMA,
        pltpu.SemaphoreType.DMA,
    ]
    scratch.append(pltpu.VMEM_SHARED((NP, D), jnp.float32))
    cp = pltpu.CompilerParams()
    if "needs_layout_passes" in pltpu.CompilerParams.__dataclass_fields__:
        cp = dataclasses.replace(cp, needs_layout_passes=False)
    return pl.kernel(
        _sc_body,
        out_type=out_type,
        mesh=_mesh,
        scratch_types=scratch,
        compiler_params=cp,
    )


_sc_pass_counts = _make_sc_pass()


R = 1000


def _t1_body(p_ref, c_ref, x_ref, wl_ref, bl_ref, wr_ref, o_ref):
    cnt = jnp.maximum(c_ref[0, :, 0:1] + c_ref[1, :, 0:1], 1.0)
    mean = (p_ref[0] + p_ref[1]) / cnt
    acc = lax.dot_general(mean, wl_ref[...], (((1,), (1,)), ((), ())),
                          preferred_element_type=jnp.float32)
    acc += lax.dot_general(x_ref[...], wr_ref[...], (((1,), (1,)), ((), ())),
                           preferred_element_type=jnp.float32)
    o_ref[...] = jnp.maximum(acc + bl_ref[...], 0.0)


def _t2_body(p_ref, c_ref, x_ref, w2l_ref, b2l_ref, w2r_ref,
             w3l_ref, b3l_ref, w3r_ref, h1_ref, h2_ref):
    cnt = jnp.maximum(c_ref[0, :, 0:1] + c_ref[1, :, 0:1], 1.0)
    mean = (p_ref[0] + p_ref[1]) / cnt
    xt = x_ref[...]
    a1 = lax.dot_general(mean, w2l_ref[...], (((1,), (1,)), ((), ())),
                         preferred_element_type=jnp.float32)
    a1 += lax.dot_general(xt, w2r_ref[...], (((1,), (1,)), ((), ())),
                          preferred_element_type=jnp.float32)
    h1_ref[...] = a1 + b2l_ref[...]
    a2 = lax.dot_general(mean, w3l_ref[...], (((1,), (1,)), ((), ())),
                         preferred_element_type=jnp.float32)
    a2 += lax.dot_general(xt, w3r_ref[...], (((1,), (1,)), ((), ())),
                          preferred_element_type=jnp.float32)
    h2_ref[...] = a2 + b3l_ref[...]


def _full(shape):
    return pl.BlockSpec(shape, lambda i: tuple(0 for _ in shape))


_p_spec = pl.BlockSpec((NC, R, D), lambda i: (0, i, 0))
_x_spec = pl.BlockSpec((R, D), lambda i: (i, 0))

_t1 = pl.pallas_call(
    _t1_body,
    grid=(N // R,),
    in_specs=[_p_spec, _p_spec, _x_spec, _full((D, D)), _full((1, D)),
              _full((D, D))],
    out_specs=_x_spec,
    out_shape=jax.ShapeDtypeStruct((N, D), jnp.float32),
)

_t2 = pl.pallas_call(
    _t2_body,
    grid=(N // R,),
    in_specs=[_p_spec, _p_spec, _x_spec, _full((D, D)), _full((1, D)),
              _full((D, D)), _full((D, D)), _full((1, D)), _full((D, D))],
    out_specs=[_x_spec, _x_spec],
    out_shape=[jax.ShapeDtypeStruct((N, D), jnp.float32),
               jax.ShapeDtypeStruct((N, D), jnp.float32)],
)


def kernel(x, edge_index, W1l, b1l, W1r, W2l, b2l, W2r, W3l, b3l, W3r):
    src = edge_index[0]
    dst = edge_index[1]
    npad = EPAD - E
    src = jnp.concatenate([src, jnp.zeros((npad,), jnp.int32)])
    pad_dst = N + (jnp.arange(npad, dtype=jnp.int32) % (NP - N))
    dst = jnp.concatenate([dst, pad_dst])
    z = jnp.zeros((CHUNK, D), jnp.float32)

    p1, cnts, _h1 = _sc_pass_counts(x, src, dst, z)
    xt = _t1(p1, cnts, x, W1l, b1l.reshape(1, D), W1r)
    p2, _c2, _h2 = _sc_pass_counts(xt, src, dst, z)
    h_, h = _t2(p2, cnts, xt, W2l, b2l.reshape(1, D), W2r,
                W3l, b3l.reshape(1, D), W3r)
    return (h_, h)


# speedup vs baseline: 1.9774x; 1.9774x over previous
import dataclasses
import functools

import jax
import jax.numpy as jnp
from jax import lax
from jax.experimental import pallas as pl
from jax.experimental.pallas import tpu as pltpu
from jax.experimental.pallas import tpu_sc as plsc

N = 10000
E = 320000
D = 128
L = 16

NC = 2
NS = 16
NW = NC * NS
CHUNK = 80
NCHUNK = 126
EW = CHUNK * NCHUNK
EPAD = NW * EW + 2 * CHUNK
NP = 10240
RT = NP // NS

_mesh = plsc.VectorSubcoreMesh(core_axis_name="c", subcore_axis_name="s")


def _sc_body(x_hbm, src_hbm, dst_hbm, z_hbm, out_hbm, cnt_hbm, hist_hbm,
             sb0, db0, sb1, db1, rows0, rows1, hist_v, tmp_v,
             si0, si1, sg0, sg1, acc_sh):
    cid = lax.axis_index("c")
    sid = lax.axis_index("s")
    wid = cid * NS + sid
    row0 = sid * RT

    sb = (sb0, sb1)
    db = (db0, db1)
    rows = (rows0, rows1)
    si = (si0, si1)
    sg = (sg0, sg1)

    pltpu.sync_copy(z_hbm, rows0)
    for k in range(RT // CHUNK):
        pltpu.sync_copy(rows0, acc_sh.at[pl.ds(row0 + k * CHUNK, CHUNK)])

    @pl.loop(0, NP // L)
    def _(i):
        hist_v[pl.ds(i * L, L)] = jnp.zeros((L,), jnp.float32)

    plsc.subcore_barrier()

    base = wid * EW

    pltpu.sync_copy(src_hbm.at[pl.ds(base, CHUNK)], sb0)
    pltpu.sync_copy(dst_hbm.at[pl.ds(base, CHUNK)], db0)
    pltpu.async_copy(src_hbm.at[pl.ds(base + CHUNK, CHUNK)], sb1, si1)
    pltpu.async_copy(dst_hbm.at[pl.ds(base + CHUNK, CHUNK)], db1, si1)
    pltpu.async_copy(x_hbm.at[sb0], rows0, sg0)

    def _phase(j, p):
        q = 1 - p
        pltpu.make_async_copy(src_hbm.at[pl.ds(base + (j + 1) * CHUNK, CHUNK)],
                              sb[q], si[q]).wait()
        pltpu.make_async_copy(dst_hbm.at[pl.ds(base + (j + 1) * CHUNK, CHUNK)],
                              db[q], si[q]).wait()
        pltpu.make_async_copy(x_hbm.at[sb[p]], rows[p], sg[p]).wait()
        pltpu.async_copy(x_hbm.at[sb[q]], rows[q], sg[q])
        pltpu.sync_copy(rows[p], acc_sh.at[db[p]], add=True)
        for k in range(CHUNK // L):
            idx = db[p][pl.ds(k * L, L)]
            cnts, last = plsc.scan_count(idx)
            plsc.addupdate_scatter(hist_v, [idx],
                                   cnts.astype(jnp.float32), mask=last)
        pltpu.async_copy(src_hbm.at[pl.ds(base + (j + 2) * CHUNK, CHUNK)],
                         sb[p], si[p])
        pltpu.async_copy(dst_hbm.at[pl.ds(base + (j + 2) * CHUNK, CHUNK)],
                         db[p], si[p])

    @pl.loop(0, NCHUNK // 2)
    def _(t):
        j = t * 2
        _phase(j, 0)
        _phase(j + 1, 1)

    pltpu.make_async_copy(src_hbm.at[pl.ds(base, CHUNK)], sb[1], si[1]).wait()
    pltpu.make_async_copy(dst_hbm.at[pl.ds(base, CHUNK)], db[1], si[1]).wait()
    pltpu.make_async_copy(x_hbm.at[sb[0]], rows[0], sg[0]).wait()

    pltpu.sync_copy(hist_v, hist_hbm.at[pl.ds(wid * NP, NP)])
    plsc.subcore_barrier()

    nko = RT // CHUNK

    def _osl(k):
        return pl.ds(row0 + k * CHUNK, CHUNK)

    pltpu.async_copy(acc_sh.at[_osl(0)], rows0, sg0)
    for k in range(nko):
        rp = rows[k % 2]
        rq = rows[1 - k % 2]
        sp = sg[k % 2]
        sq = sg[1 - k % 2]
        wp = si[k % 2]
        wq = si[1 - k % 2]
        pltpu.make_async_copy(acc_sh.at[_osl(k)], rp, sp).wait()
        if k + 1 < nko:
            if k >= 1:
                pltpu.make_async_copy(rq, out_hbm.at[cid, _osl(k - 1)],
                                      wq).wait()
            pltpu.async_copy(acc_sh.at[_osl(k + 1)], rq, sq)
        pltpu.async_copy(rp, out_hbm.at[cid, _osl(k)], wp)
    pltpu.make_async_copy(rows[(nko - 2) % 2],
                          out_hbm.at[cid, _osl(nko - 2)],
                          si[(nko - 2) % 2]).wait()
    pltpu.make_async_copy(rows[(nko - 1) % 2],
                          out_hbm.at[cid, _osl(nko - 1)],
                          si[(nko - 1) % 2]).wait()

    lane_iota = jax.lax.iota(jnp.int32, L)
    zeros_i = jnp.zeros((L,), jnp.int32)

    def _hsl(j, k):
        return pl.ds((cid * NS + j) * NP + row0 + k * CHUNK, CHUNK)

    def _fire(k):
        for j in range(NS):
            pltpu.async_copy(hist_hbm.at[_hsl(j, k)],
                             tmp_v.at[pl.ds(j * CHUNK, CHUNK)], sg0)

    def _drain(k):
        for j in range(NS):
            pltpu.make_async_copy(hist_hbm.at[_hsl(j, k)],
                                  tmp_v.at[pl.ds(j * CHUNK, CHUNK)],
                                  sg0).wait()

    _fire(0)
    for k in range(RT // CHUNK):
        _drain(k)

        @pl.loop(0, CHUNK // L)
        def _(b):
            tot = jnp.zeros((L,), jnp.float32)
            for j in range(NS):
                tot += tmp_v[pl.ds(j * CHUNK + b * L, L)]
            plsc.store_scatter(rows0, [b * L + lane_iota, zeros_i], tot)

        if k + 1 < RT // CHUNK:
            _fire(k + 1)
        pltpu.sync_copy(rows0, cnt_hbm.at[cid, _osl(k)])


def _make_sc_pass():
    out_type = (jax.ShapeDtypeStruct((NC, NP, D), jnp.float32),
                jax.ShapeDtypeStruct((NC, NP, D), jnp.float32),
                jax.ShapeDtypeStruct((NW * NP,), jnp.float32))
    scratch = [
        pltpu.VMEM((CHUNK,), jnp.int32),
        pltpu.VMEM((CHUNK,), jnp.int32),
        pltpu.VMEM((CHUNK,), jnp.int32),
        pltpu.VMEM((CHUNK,), jnp.int32),
        pltpu.VMEM((CHUNK, D), jnp.float32),
        pltpu.VMEM((CHUNK, D), jnp.float32),
        pltpu.VMEM((NP,), jnp.float32),
        pltpu.VMEM((NS * CHUNK,), jnp.float32),
        pltpu.SemaphoreType.DMA,
        pltpu.SemaphoreType.DMA,
        pltpu.SemaphoreType.DMA,
        pltpu.SemaphoreType.DMA,
    ]
    scratch.append(pltpu.VMEM_SHARED((NP, D), jnp.float32))
    cp = pltpu.CompilerParams()
    if "needs_layout_passes" in pltpu.CompilerParams.__dataclass_fields__:
        cp = dataclasses.replace(cp, needs_layout_passes=False)
    return pl.kernel(
        _sc_body,
        out_type=out_type,
        mesh=_mesh,
        scratch_types=scratch,
        compiler_params=cp,
    )


_sc_pass_counts = _make_sc_pass()


R = 1000


def _t1_body(p_ref, c_ref, x_ref, wl_ref, bl_ref, wr_ref, o_ref):
    cnt = jnp.maximum(c_ref[0, :, 0:1] + c_ref[1, :, 0:1], 1.0)
    mean = (p_ref[0] + p_ref[1]) / cnt
    acc = lax.dot_general(mean, wl_ref[...], (((1,), (1,)), ((), ())),
                          preferred_element_type=jnp.float32)
    acc += lax.dot_general(x_ref[...], wr_ref[...], (((1,), (1,)), ((), ())),
                           preferred_element_type=jnp.float32)
    o_ref[...] = jnp.maximum(acc + bl_ref[...], 0.0)


def _t2_body(p_ref, c_ref, x_ref, w2l_ref, b2l_ref, w2r_ref,
             w3l_ref, b3l_ref, w3r_ref, h1_ref, h2_ref):
    cnt = jnp.maximum(c_ref[0, :, 0:1] + c_ref[1, :, 0:1], 1.0)
    mean = (p_ref[0] + p_ref[1]) / cnt
    xt = x_ref[...]
    a1 = lax.dot_general(mean, w2l_ref[...], (((1,), (1,)), ((), ())),
                         preferred_element_type=jnp.float32)
    a1 += lax.dot_general(xt, w2r_ref[...], (((1,), (1,)), ((), ())),
                          preferred_element_type=jnp.float32)
    h1_ref[...] = a1 + b2l_ref[...]
    a2 = lax.dot_general(mean, w3l_ref[...], (((1,), (1,)), ((), ())),
                         preferred_element_type=jnp.float32)
    a2 += lax.dot_general(xt, w3r_ref[...], (((1,), (1,)), ((), ())),
                          preferred_element_type=jnp.float32)
    h2_ref[...] = a2 + b3l_ref[...]


def _full(shape):
    return pl.BlockSpec(shape, lambda i: tuple(0 for _ in shape))


_p_spec = pl.BlockSpec((NC, R, D), lambda i: (0, i, 0))
_x_spec = pl.BlockSpec((R, D), lambda i: (i, 0))

_t1 = pl.pallas_call(
    _t1_body,
    grid=(N // R,),
    in_specs=[_p_spec, _p_spec, _x_spec, _full((D, D)), _full((1, D)),
              _full((D, D))],
    out_specs=_x_spec,
    out_shape=jax.ShapeDtypeStruct((N, D), jnp.float32),
)

_t2 = pl.pallas_call(
    _t2_body,
    grid=(N // R,),
    in_specs=[_p_spec, _p_spec, _x_spec, _full((D, D)), _full((1, D)),
              _full((D, D)), _full((D, D)), _full((1, D)), _full((D, D))],
    out_specs=[_x_spec, _x_spec],
    out_shape=[jax.ShapeDtypeStruct((N, D), jnp.float32),
               jax.ShapeDtypeStruct((N, D), jnp.float32)],
)


def kernel(x, edge_index, W1l, b1l, W1r, W2l, b2l, W2r, W3l, b3l, W3r):
    src = edge_index[0]
    dst = edge_index[1]
    npad = EPAD - E
    src = jnp.concatenate([src, jnp.zeros((npad,), jnp.int32)])
    pad_dst = N + (jnp.arange(npad, dtype=jnp.int32) % (NP - N))
    dst = jnp.concatenate([dst, pad_dst])
    z = jnp.zeros((CHUNK, D), jnp.float32)

    p1, cnts, _h1 = _sc_pass_counts(x, src, dst, z)
    xt = _t1(p1, cnts, x, W1l, b1l.reshape(1, D), W1r)
    p2, _c2, _h2 = _sc_pass_counts(xt, src, dst, z)
    h_, h = _t2(p2, cnts, xt, W2l, b2l.reshape(1, D), W2r,
                W3l, b3l.reshape(1, D), W3r)
    return (h_, h)
